# Initial kernel scaffold; baseline (speedup 1.0000x reference)
import jax
import jax.numpy as jnp
from jax.experimental import pallas as pl


def kernel(x, frame_embed, beat_embed, bar_embed, pe):
    raise NotImplementedError("write your pallas kernel here")


# speedup vs baseline: 2.5588x; 2.5588x over previous
import jax
import jax.numpy as jnp
from jax.experimental import pallas as pl

D_MODEL = 1024
FPB = 43
BPB = 4
BPP = 4
DPS = D_MODEL // 4
BS = 512


def _add_pe_kernel(fe_ref, be_ref, ba_ref, x_ref, pe_ref, o_ref):
    j = pl.program_id(0)
    row = j * BS + jax.lax.broadcasted_iota(jnp.int32, (BS, 1), 0)
    beat_pos = row % FPB
    bar_pos = (row // FPB) % BPB
    phrase_pos = (row // (FPB * BPB)) % BPP
    cols48 = jax.lax.broadcasted_iota(jnp.int32, (BS, 48), 1)
    cols8 = jax.lax.broadcasted_iota(jnp.int32, (BS, 8), 1)
    oh_f = (cols48 == beat_pos).astype(jnp.float32)
    oh_b = (cols8 == bar_pos).astype(jnp.float32)
    oh_p = (cols8 == phrase_pos).astype(jnp.float32)
    f = jnp.dot(oh_f, fe_ref[...], preferred_element_type=jnp.float32)
    b = jnp.dot(oh_b, be_ref[...], preferred_element_type=jnp.float32)
    p = jnp.dot(oh_p, ba_ref[...], preferred_element_type=jnp.float32)
    enc = jnp.concatenate([f, b, p, pe_ref[...]], axis=-1)
    o_ref[...] = x_ref[...] + enc[None, :, :]


def kernel(x, frame_embed, beat_embed, bar_embed, pe):
    B, S, D = x.shape
    fe = jnp.zeros((48, DPS), x.dtype).at[:FPB].set(frame_embed)
    be = jnp.zeros((8, DPS), x.dtype).at[:BPB].set(beat_embed)
    ba = jnp.zeros((8, DPS), x.dtype).at[:BPP].set(bar_embed)
    return pl.pallas_call(
        _add_pe_kernel,
        grid=(S // BS,),
        in_specs=[
            pl.BlockSpec((48, DPS), lambda j: (0, 0)),
            pl.BlockSpec((8, DPS), lambda j: (0, 0)),
            pl.BlockSpec((8, DPS), lambda j: (0, 0)),
            pl.BlockSpec((B, BS, D), lambda j: (0, j, 0)),
            pl.BlockSpec((BS, DPS), lambda j: (j, 0)),
        ],
        out_specs=pl.BlockSpec((B, BS, D), lambda j: (0, j, 0)),
        out_shape=jax.ShapeDtypeStruct((B, S, D), x.dtype),
    )(fe, be, ba, x, pe)
